# baseline (speedup 1.0000x reference)
import functools

import jax
import jax.numpy as jnp
from jax import lax
from jax.experimental import pallas as pl
from jax.experimental.pallas import tpu as pltpu
from jax.experimental.pallas import tpu_sc as plsc

N_NODE = 5000
D = 128
E = 160000

NC = 2
NS = 16
K = 80
EPT = E // NS
NCH = EPT // K
NBUF = 5
ACC_ROWS = 5120
STRIPE = ACC_ROWS // NS

_F32 = jnp.float32


def _zero_vmem_2d(ref, nrows):
    def body(r, carry):
        for k in range(D // 16):
            ref[r, pl.ds(k * 16, 16)] = jnp.zeros((16,), _F32)
        return carry
    lax.fori_loop(0, nrows, body, None)


def _fill_vmem_1d(ref, n, val):
    def body(k, carry):
        ref[pl.ds(k * 16, 16)] = jnp.full((16,), val, _F32)
        return carry
    lax.fori_loop(0, n // 16, body, None)


def _agg_one_type(with_counts, t, obase, x_hbm, sidx_hbm, didx_hbm,
                  out_s_hbm, out_c_hbm, acc, cacc, bufs,
                  sidx, didx, ones_v, zc, gsems, ssems, osem):
    _zero_vmem_2d(bufs[0], K)
    base = t * STRIPE
    for i in range(STRIPE // K):
        pltpu.sync_copy(bufs[0], acc.at[pl.ds(base + i * K, K)])
    if with_counts:
        _fill_vmem_1d(zc, STRIPE, 0.0)
        pltpu.sync_copy(zc, cacc.at[pl.ds(base, STRIPE)])
        _fill_vmem_1d(ones_v, K, 1.0)
    pltpu.sync_copy(sidx_hbm.at[t], sidx)
    pltpu.sync_copy(didx_hbm.at[t], didx)
    plsc.subcore_barrier()

    for i in range(NBUF - 1):
        pltpu.async_copy(x_hbm.at[sidx.at[i]], bufs[i], gsems[i])

    def group(g, carry):
        c0 = g * NBUF
        for i in range(NBUF):
            c = c0 + i
            rc = c + NBUF - 1
            rslot = (i + NBUF - 1) % NBUF
            @pl.when(rc >= NBUF)
            def _():
                pltpu.make_async_copy(bufs[rslot], acc.at[didx.at[0]],
                                      ssems[rslot]).wait()
            @pl.when(rc < NCH)
            def _():
                pltpu.async_copy(x_hbm.at[sidx.at[rc]], bufs[rslot],
                                 gsems[rslot])
            pltpu.make_async_copy(x_hbm.at[sidx.at[c]], bufs[i],
                                  gsems[i]).wait()
            pltpu.async_copy(bufs[i], acc.at[didx.at[c]], ssems[i], add=True)
            if with_counts:
                pltpu.async_copy(ones_v, cacc.at[didx.at[c]], osem, add=True)
        return carry

    lax.fori_loop(0, NCH // NBUF, group, None)
    pltpu.make_async_copy(bufs[(NCH - 1) % NBUF], acc.at[didx.at[0]],
                          ssems[(NCH - 1) % NBUF]).wait()
    if with_counts:
        def drain(j, carry):
            pltpu.make_async_copy(ones_v, cacc.at[didx.at[0]], osem).wait()
            return carry
        lax.fori_loop(0, NCH, drain, None)

    plsc.subcore_barrier()
    last = N_NODE - (NS - 1) * STRIPE
    if with_counts:
        pltpu.sync_copy(cacc.at[pl.ds(base, STRIPE)], zc)
    @pl.when(t < NS - 1)
    def _():
        pltpu.sync_copy(acc.at[pl.ds(base, STRIPE)],
                        out_s_hbm.at[pl.ds(obase + base, STRIPE)])
        if with_counts:
            pltpu.sync_copy(zc, out_c_hbm.at[pl.ds(obase + base, STRIPE)])
    @pl.when(t == NS - 1)
    def _():
        pltpu.sync_copy(acc.at[pl.ds(base, last)],
                        out_s_hbm.at[pl.ds(obase + base, last)])
        if with_counts:
            pltpu.sync_copy(zc.at[pl.ds(0, last)],
                            out_c_hbm.at[pl.ds(obase + base, last)])


_SC_MESH = plsc.VectorSubcoreMesh(
    core_axis_name="c", subcore_axis_name="s", num_cores=NC, num_subcores=NS)


def _make_sc_aggregate(with_counts):
    out_type = [jax.ShapeDtypeStruct((2 * N_NODE, D), _F32)]
    if with_counts:
        out_type += [jax.ShapeDtypeStruct((2 * N_NODE,), _F32)]

    @functools.partial(
        pl.kernel,
        out_type=out_type,
        mesh=_SC_MESH,
        scratch_types=[
            pltpu.VMEM_SHARED((ACC_ROWS, D), _F32),
            pltpu.VMEM_SHARED((ACC_ROWS,), _F32),
            [pltpu.VMEM((K, D), _F32)] * NBUF,
            pltpu.VMEM((NCH, K), jnp.int32),
            pltpu.VMEM((NCH, K), jnp.int32),
            pltpu.VMEM((K,), _F32),
            pltpu.VMEM((STRIPE,), _F32),
            [pltpu.SemaphoreType.DMA] * NBUF,
            [pltpu.SemaphoreType.DMA] * NBUF,
            pltpu.SemaphoreType.DMA,
        ],
    )
    def _sc_aggregate(x_hbm, sidx_u2i, didx_u2i, sidx_i2u, didx_i2u,
                      s_hbm, *rest):
        if with_counts:
            c_hbm = rest[0]
            rest = rest[1:]
        else:
            c_hbm = None
        (acc, cacc, bufs, sidx, didx, ones_v, zc,
         gsems, ssems, osem) = rest
        c = lax.axis_index("c")
        t = lax.axis_index("s")

        @pl.when(c == 0)
        def _():
            _agg_one_type(with_counts, t, 0, x_hbm, sidx_u2i, didx_u2i,
                          s_hbm, c_hbm, acc, cacc, bufs,
                          sidx, didx, ones_v, zc, gsems, ssems, osem)

        @pl.when(c == 1)
        def _():
            _agg_one_type(with_counts, t, N_NODE, x_hbm, sidx_i2u, didx_i2u,
                          s_hbm, c_hbm, acc, cacc, bufs,
                          sidx, didx, ones_v, zc, gsems, ssems, osem)

    return _sc_aggregate


_sc_aggregate_l1 = _make_sc_aggregate(with_counts=True)
_sc_aggregate_l2 = _make_sc_aggregate(with_counts=False)


BLK = 1000
NBLK = 2 * N_NODE // BLK


def _self_body(x_ref, ws_ref, b_ref, o_ref):
    o_ref[...] = jnp.dot(x_ref[...], ws_ref[0],
                         preferred_element_type=jnp.float32) + b_ref[0]


def _tc_self(x, ws, b):
    half = NBLK // 2
    return pl.pallas_call(
        _self_body,
        grid=(NBLK,),
        in_specs=[
            pl.BlockSpec((BLK, D), lambda i: (i, 0)),
            pl.BlockSpec((1, D, D), lambda i: (i // half, 0, 0)),
            pl.BlockSpec((1, 1, D), lambda i: (i // half, 0, 0)),
        ],
        out_specs=pl.BlockSpec((BLK, D), lambda i: (i, 0)),
        out_shape=jax.ShapeDtypeStruct((2 * N_NODE, D), jnp.float32),
    )(x, ws, b.reshape(2, 1, D))


def _combine_body(relu, s_ref, c_ref, y_ref, wn_ref, o_ref):
    cnt = jnp.maximum(c_ref[...], 1.0)
    agg = s_ref[...] / cnt
    y = jnp.dot(agg, wn_ref[0], preferred_element_type=jnp.float32)
    y = y + y_ref[...]
    if relu:
        y = jnp.maximum(y, 0.0)
    o_ref[...] = y


def _tc_combine(s, cnt, y_self, wn, relu, rotate_out):
    half = NBLK // 2
    out_map = (lambda i: ((i + half) % NBLK, 0)) if rotate_out \
        else (lambda i: (i, 0))
    return pl.pallas_call(
        functools.partial(_combine_body, relu),
        grid=(NBLK,),
        in_specs=[
            pl.BlockSpec((BLK, D), lambda i: (i, 0)),
            pl.BlockSpec((BLK, 1), lambda i: (i, 0)),
            pl.BlockSpec((BLK, D), lambda i: (i, 0)),
            pl.BlockSpec((1, D, D), lambda i: (i // half, 0, 0)),
        ],
        out_specs=pl.BlockSpec((BLK, D), out_map),
        out_shape=jax.ShapeDtypeStruct((2 * N_NODE, D), jnp.float32),
    )(s, cnt.reshape(2 * N_NODE, 1), y_self, wn)


def kernel(x_user, x_item, w_neigh_u2i_1, w_self_u2i_1, b_u2i_1,
           w_neigh_i2u_1, w_self_i2u_1, b_i2u_1,
           w_neigh_u2i_2, w_self_u2i_2, b_u2i_2,
           w_neigh_i2u_2, w_self_i2u_2, b_i2u_2,
           edge_index_u2i, edge_index_i2u):
    su = edge_index_u2i.astype(jnp.int32)
    si = edge_index_i2u.astype(jnp.int32)
    sidx_u = (su[0] + N_NODE).reshape(NS, NCH, K)
    didx_u = su[1].reshape(NS, NCH, K)
    sidx_i = si[0].reshape(NS, NCH, K)
    didx_i = si[1].reshape(NS, NCH, K)
    x_all = jnp.concatenate([x_item, x_user])
    wn1 = jnp.stack([w_neigh_u2i_1, w_neigh_i2u_1])
    ws1 = jnp.stack([w_self_u2i_1, w_self_i2u_1])
    b1 = jnp.stack([b_u2i_1, b_i2u_1])
    wn2 = jnp.stack([w_neigh_u2i_2, w_neigh_i2u_2])
    ws2 = jnp.stack([w_self_u2i_2, w_self_i2u_2])
    b2 = jnp.stack([b_u2i_2, b_i2u_2])

    s1, c1 = _sc_aggregate_l1(x_all, sidx_u, didx_u, sidx_i, didx_i)
    y1 = _tc_self(x_all, ws1, b1)
    h_all = _tc_combine(s1, c1, y1, wn1, relu=True, rotate_out=False)

    (s2,) = _sc_aggregate_l2(h_all, sidx_u, didx_u, sidx_i, didx_i)
    y2 = _tc_self(h_all, ws2, b2)
    return _tc_combine(s2, c1, y2, wn2, relu=False, rotate_out=True)


# speedup vs baseline: 1.0094x; 1.0094x over previous
import functools

import jax
import jax.numpy as jnp
from jax import lax
from jax.experimental import pallas as pl
from jax.experimental.pallas import tpu as pltpu
from jax.experimental.pallas import tpu_sc as plsc

N_NODE = 5000
D = 128
E = 160000

NC = 2
NS = 16
K = 80
EPT = E // NS
NCH = EPT // K
NBUF = 5
ACC_ROWS = 5120
STRIPE = ACC_ROWS // NS

_F32 = jnp.float32


def _zero_vmem_2d(ref, nrows):
    def body(r, carry):
        for k in range(D // 16):
            ref[r, pl.ds(k * 16, 16)] = jnp.zeros((16,), _F32)
        return carry
    lax.fori_loop(0, nrows, body, None)


def _fill_vmem_1d(ref, n, val):
    def body(k, carry):
        ref[pl.ds(k * 16, 16)] = jnp.full((16,), val, _F32)
        return carry
    lax.fori_loop(0, n // 16, body, None)


def _agg_one_type(with_counts, t, obase, x_hbm, sidx_hbm, didx_hbm,
                  out_s_hbm, out_c_hbm, acc, cacc, bufs,
                  sidx, didx, ones_v, zc, gsems, ssems, osem):
    _zero_vmem_2d(bufs[0], K)
    base = t * STRIPE
    for i in range(STRIPE // K):
        pltpu.sync_copy(bufs[0], acc.at[pl.ds(base + i * K, K)])
    if with_counts:
        _fill_vmem_1d(zc, STRIPE, 0.0)
        pltpu.sync_copy(zc, cacc.at[pl.ds(base, STRIPE)])
        _fill_vmem_1d(ones_v, K, 1.0)
    pltpu.sync_copy(sidx_hbm.at[t], sidx)
    pltpu.sync_copy(didx_hbm.at[t], didx)
    plsc.subcore_barrier()

    for i in range(NBUF - 1):
        pltpu.async_copy(x_hbm.at[sidx.at[i]], bufs[i], gsems[i])

    def group(g, carry):
        c0 = g * NBUF
        for i in range(NBUF):
            c = c0 + i
            rc = c + NBUF - 1
            rslot = (i + NBUF - 1) % NBUF
            @pl.when(rc >= NBUF)
            def _():
                pltpu.make_async_copy(bufs[rslot], acc.at[didx.at[0]],
                                      ssems[rslot]).wait()
            @pl.when(rc < NCH)
            def _():
                pltpu.async_copy(x_hbm.at[sidx.at[rc]], bufs[rslot],
                                 gsems[rslot])
            pltpu.make_async_copy(x_hbm.at[sidx.at[c]], bufs[i],
                                  gsems[i]).wait()
            pltpu.async_copy(bufs[i], acc.at[didx.at[c]], ssems[i], add=True)
            if with_counts:
                pltpu.async_copy(ones_v, cacc.at[didx.at[c]], osem, add=True)
        return carry

    lax.fori_loop(0, NCH // NBUF, group, None)
    pltpu.make_async_copy(bufs[(NCH - 1) % NBUF], acc.at[didx.at[0]],
                          ssems[(NCH - 1) % NBUF]).wait()
    if with_counts:
        def drain(j, carry):
            pltpu.make_async_copy(ones_v, cacc.at[didx.at[0]], osem).wait()
            return carry
        lax.fori_loop(0, NCH, drain, None)

    plsc.subcore_barrier()
    last = N_NODE - (NS - 1) * STRIPE
    if with_counts:
        pltpu.sync_copy(cacc.at[pl.ds(base, STRIPE)], zc)
    @pl.when(t < NS - 1)
    def _():
        pltpu.sync_copy(acc.at[pl.ds(base, STRIPE)],
                        out_s_hbm.at[pl.ds(obase + base, STRIPE)])
        if with_counts:
            pltpu.sync_copy(zc, out_c_hbm.at[pl.ds(obase + base, STRIPE)])
    @pl.when(t == NS - 1)
    def _():
        pltpu.sync_copy(acc.at[pl.ds(base, last)],
                        out_s_hbm.at[pl.ds(obase + base, last)])
        if with_counts:
            pltpu.sync_copy(zc.at[pl.ds(0, last)],
                            out_c_hbm.at[pl.ds(obase + base, last)])


_SC_MESH = plsc.VectorSubcoreMesh(
    core_axis_name="c", subcore_axis_name="s", num_cores=NC, num_subcores=NS)


def _make_sc_aggregate(with_counts):
    out_type = [jax.ShapeDtypeStruct((2 * N_NODE, D), _F32)]
    if with_counts:
        out_type += [jax.ShapeDtypeStruct((2 * N_NODE,), _F32)]

    @functools.partial(
        pl.kernel,
        out_type=out_type,
        mesh=_SC_MESH,
        scratch_types=[
            pltpu.VMEM_SHARED((ACC_ROWS, D), _F32),
            pltpu.VMEM_SHARED((ACC_ROWS,), _F32),
            [pltpu.VMEM((K, D), _F32)] * NBUF,
            pltpu.VMEM((NCH, K), jnp.int32),
            pltpu.VMEM((NCH, K), jnp.int32),
            pltpu.VMEM((K,), _F32),
            pltpu.VMEM((STRIPE,), _F32),
            [pltpu.SemaphoreType.DMA] * NBUF,
            [pltpu.SemaphoreType.DMA] * NBUF,
            pltpu.SemaphoreType.DMA,
        ],
    )
    def _sc_aggregate(x_hbm, sidx_u2i, didx_u2i, sidx_i2u, didx_i2u,
                      s_hbm, *rest):
        if with_counts:
            c_hbm = rest[0]
            rest = rest[1:]
        else:
            c_hbm = None
        (acc, cacc, bufs, sidx, didx, ones_v, zc,
         gsems, ssems, osem) = rest
        c = lax.axis_index("c")
        t = lax.axis_index("s")

        @pl.when(c == 0)
        def _():
            _agg_one_type(with_counts, t, 0, x_hbm, sidx_u2i, didx_u2i,
                          s_hbm, c_hbm, acc, cacc, bufs,
                          sidx, didx, ones_v, zc, gsems, ssems, osem)

        @pl.when(c == 1)
        def _():
            _agg_one_type(with_counts, t, N_NODE, x_hbm, sidx_i2u, didx_i2u,
                          s_hbm, c_hbm, acc, cacc, bufs,
                          sidx, didx, ones_v, zc, gsems, ssems, osem)

    return _sc_aggregate


_sc_aggregate_l1 = _make_sc_aggregate(with_counts=True)
_sc_aggregate_l2 = _make_sc_aggregate(with_counts=False)


BLK = 1000
NBLK = 2 * N_NODE // BLK


def _linear_body(relu, s_ref, c_ref, x_ref, wn_ref, ws_ref, b_ref, o_ref):
    cnt = jnp.maximum(c_ref[...], 1.0)
    agg = s_ref[...] / cnt
    y = jnp.dot(agg, wn_ref[0], preferred_element_type=jnp.float32)
    y = y + jnp.dot(x_ref[...], ws_ref[0], preferred_element_type=jnp.float32)
    y = y + b_ref[0]
    if relu:
        y = jnp.maximum(y, 0.0)
    o_ref[...] = y


def _tc_linear(s, cnt, x, wn, ws, b, relu, rotate_out):
    half = NBLK // 2
    out_map = (lambda i: ((i + half) % NBLK, 0)) if rotate_out \
        else (lambda i: (i, 0))
    return pl.pallas_call(
        functools.partial(_linear_body, relu),
        grid=(NBLK,),
        in_specs=[
            pl.BlockSpec((BLK, D), lambda i: (i, 0)),
            pl.BlockSpec((BLK, 1), lambda i: (i, 0)),
            pl.BlockSpec((BLK, D), lambda i: (i, 0)),
            pl.BlockSpec((1, D, D), lambda i: (i // half, 0, 0)),
            pl.BlockSpec((1, D, D), lambda i: (i // half, 0, 0)),
            pl.BlockSpec((1, 1, D), lambda i: (i // half, 0, 0)),
        ],
        out_specs=pl.BlockSpec((BLK, D), out_map),
        out_shape=jax.ShapeDtypeStruct((2 * N_NODE, D), jnp.float32),
    )(s, cnt.reshape(2 * N_NODE, 1), x, wn, ws, b.reshape(2, 1, D))


def kernel(x_user, x_item, w_neigh_u2i_1, w_self_u2i_1, b_u2i_1,
           w_neigh_i2u_1, w_self_i2u_1, b_i2u_1,
           w_neigh_u2i_2, w_self_u2i_2, b_u2i_2,
           w_neigh_i2u_2, w_self_i2u_2, b_i2u_2,
           edge_index_u2i, edge_index_i2u):
    su = edge_index_u2i.astype(jnp.int32)
    si = edge_index_i2u.astype(jnp.int32)
    sidx_u = (su[0] + N_NODE).reshape(NS, NCH, K)
    didx_u = su[1].reshape(NS, NCH, K)
    sidx_i = si[0].reshape(NS, NCH, K)
    didx_i = si[1].reshape(NS, NCH, K)
    x_all = jnp.concatenate([x_item, x_user])
    wn1 = jnp.stack([w_neigh_u2i_1, w_neigh_i2u_1])
    ws1 = jnp.stack([w_self_u2i_1, w_self_i2u_1])
    b1 = jnp.stack([b_u2i_1, b_i2u_1])
    wn2 = jnp.stack([w_neigh_u2i_2, w_neigh_i2u_2])
    ws2 = jnp.stack([w_self_u2i_2, w_self_i2u_2])
    b2 = jnp.stack([b_u2i_2, b_i2u_2])

    s1, c1 = _sc_aggregate_l1(x_all, sidx_u, didx_u, sidx_i, didx_i)
    h_all = _tc_linear(s1, c1, x_all, wn1, ws1, b1,
                       relu=True, rotate_out=False)

    (s2,) = _sc_aggregate_l2(h_all, sidx_u, didx_u, sidx_i, didx_i)
    return _tc_linear(s2, c1, h_all, wn2, ws2, b2,
                      relu=False, rotate_out=True)


# speedup vs baseline: 1.0434x; 1.0337x over previous
import functools

import jax
import jax.numpy as jnp
from jax import lax
from jax.experimental import pallas as pl
from jax.experimental.pallas import tpu as pltpu
from jax.experimental.pallas import tpu_sc as plsc

N_NODE = 5000
D = 128
E = 160000

NC = 2
NS = 16
K = 80
EPT = E // NS
NCH = EPT // K
NBUF = 5
ACC_ROWS = 5120
STRIPE = ACC_ROWS // NS

_F32 = jnp.float32


def _zero_vmem_2d(ref, nrows):
    def body(r, carry):
        for k in range(D // 16):
            ref[r, pl.ds(k * 16, 16)] = jnp.zeros((16,), _F32)
        return carry
    lax.fori_loop(0, nrows, body, None)


def _fill_vmem_1d(ref, n, val):
    def body(k, carry):
        ref[pl.ds(k * 16, 16)] = jnp.full((16,), val, _F32)
        return carry
    lax.fori_loop(0, n // 16, body, None)


def _agg_one_type(with_counts, t, obase, x_hbm, sidx_hbm, didx_hbm,
                  out_s_hbm, out_c_hbm, acc, cacc, bufs,
                  sidx, didx, ones_v, zc, gsems, ssems, osem):
    _zero_vmem_2d(bufs[0], K)
    base = t * STRIPE
    for i in range(STRIPE // K):
        pltpu.sync_copy(bufs[0], acc.at[pl.ds(base + i * K, K)])
    if with_counts:
        _fill_vmem_1d(zc, STRIPE, 0.0)
        pltpu.sync_copy(zc, cacc.at[pl.ds(base, STRIPE)])
        _fill_vmem_1d(ones_v, K, 1.0)
    pltpu.sync_copy(sidx_hbm.at[t], sidx)
    pltpu.sync_copy(didx_hbm.at[t], didx)
    plsc.subcore_barrier()

    for i in range(NBUF - 1):
        pltpu.async_copy(x_hbm.at[sidx.at[i]], bufs[i], gsems[i])

    def group(g, carry):
        c0 = g * NBUF
        for i in range(NBUF):
            c = c0 + i
            rc = c + NBUF - 1
            rslot = (i + NBUF - 1) % NBUF
            @pl.when(rc >= NBUF)
            def _():
                pltpu.make_async_copy(bufs[rslot], acc.at[didx.at[0]],
                                      ssems[rslot]).wait()
            @pl.when(rc < NCH)
            def _():
                pltpu.async_copy(x_hbm.at[sidx.at[rc]], bufs[rslot],
                                 gsems[rslot])
            pltpu.make_async_copy(x_hbm.at[sidx.at[c]], bufs[i],
                                  gsems[i]).wait()
            pltpu.async_copy(bufs[i], acc.at[didx.at[c]], ssems[i], add=True)
            if with_counts:
                pltpu.async_copy(ones_v, cacc.at[didx.at[c]], osem, add=True)
        return carry

    lax.fori_loop(0, NCH // NBUF, group, None)
    pltpu.make_async_copy(bufs[(NCH - 1) % NBUF], acc.at[didx.at[0]],
                          ssems[(NCH - 1) % NBUF]).wait()
    if with_counts:
        def drain(j, carry):
            pltpu.make_async_copy(ones_v, cacc.at[didx.at[0]], osem).wait()
            return carry
        lax.fori_loop(0, NCH, drain, None)

    plsc.subcore_barrier()
    last = N_NODE - (NS - 1) * STRIPE
    if with_counts:
        pltpu.sync_copy(cacc.at[pl.ds(base, STRIPE)], zc)
    @pl.when(t < NS - 1)
    def _():
        pltpu.sync_copy(acc.at[pl.ds(base, STRIPE)],
                        out_s_hbm.at[pl.ds(obase + base, STRIPE)])
        if with_counts:
            pltpu.sync_copy(zc, out_c_hbm.at[pl.ds(obase + base, STRIPE)])
    @pl.when(t == NS - 1)
    def _():
        pltpu.sync_copy(acc.at[pl.ds(base, last)],
                        out_s_hbm.at[pl.ds(obase + base, last)])
        if with_counts:
            pltpu.sync_copy(zc.at[pl.ds(0, last)],
                            out_c_hbm.at[pl.ds(obase + base, last)])


_SC_MESH = plsc.VectorSubcoreMesh(
    core_axis_name="c", subcore_axis_name="s", num_cores=NC, num_subcores=NS)


def _make_sc_aggregate(with_counts):
    out_type = [jax.ShapeDtypeStruct((2 * N_NODE, D), _F32)]
    if with_counts:
        out_type += [jax.ShapeDtypeStruct((2 * N_NODE,), _F32)]

    @functools.partial(
        pl.kernel,
        out_type=out_type,
        mesh=_SC_MESH,
        scratch_types=[
            pltpu.VMEM_SHARED((ACC_ROWS, D), _F32),
            pltpu.VMEM_SHARED((ACC_ROWS,), _F32),
            [pltpu.VMEM((K, D), _F32)] * NBUF,
            pltpu.VMEM((NCH, K), jnp.int32),
            pltpu.VMEM((NCH, K), jnp.int32),
            pltpu.VMEM((K,), _F32),
            pltpu.VMEM((STRIPE,), _F32),
            [pltpu.SemaphoreType.DMA] * NBUF,
            [pltpu.SemaphoreType.DMA] * NBUF,
            pltpu.SemaphoreType.DMA,
        ],
    )
    def _sc_aggregate(x_hbm, sidx_u2i, didx_u2i, sidx_i2u, didx_i2u,
                      s_hbm, *rest):
        if with_counts:
            c_hbm = rest[0]
            rest = rest[1:]
        else:
            c_hbm = None
        (acc, cacc, bufs, sidx, didx, ones_v, zc,
         gsems, ssems, osem) = rest
        c = lax.axis_index("c")
        t = lax.axis_index("s")

        @pl.when(c == 0)
        def _():
            _agg_one_type(with_counts, t, 0, x_hbm, sidx_u2i, didx_u2i,
                          s_hbm, c_hbm, acc, cacc, bufs,
                          sidx, didx, ones_v, zc, gsems, ssems, osem)

        @pl.when(c == 1)
        def _():
            _agg_one_type(with_counts, t, N_NODE, x_hbm, sidx_i2u, didx_i2u,
                          s_hbm, c_hbm, acc, cacc, bufs,
                          sidx, didx, ones_v, zc, gsems, ssems, osem)

    return _sc_aggregate


_sc_aggregate_l1 = _make_sc_aggregate(with_counts=True)
_sc_aggregate_l2 = _make_sc_aggregate(with_counts=False)


BLK = 5000
NBLK = 2 * N_NODE // BLK


def _linear_body(relu, s_ref, c_ref, x_ref, wn_ref, ws_ref, b_ref, o_ref):
    cnt = jnp.maximum(c_ref[...], 1.0)
    agg = s_ref[...] / cnt
    y = jnp.dot(agg, wn_ref[0], preferred_element_type=jnp.float32)
    y = y + jnp.dot(x_ref[...], ws_ref[0], preferred_element_type=jnp.float32)
    y = y + b_ref[0]
    if relu:
        y = jnp.maximum(y, 0.0)
    o_ref[...] = y


def _tc_linear(s, cnt, x, wn, ws, b, relu, rotate_out):
    half = NBLK // 2
    out_map = (lambda i: ((i + half) % NBLK, 0)) if rotate_out \
        else (lambda i: (i, 0))
    return pl.pallas_call(
        functools.partial(_linear_body, relu),
        grid=(NBLK,),
        in_specs=[
            pl.BlockSpec((BLK, D), lambda i: (i, 0)),
            pl.BlockSpec((BLK, 1), lambda i: (i, 0)),
            pl.BlockSpec((BLK, D), lambda i: (i, 0)),
            pl.BlockSpec((1, D, D), lambda i: (i // half, 0, 0)),
            pl.BlockSpec((1, D, D), lambda i: (i // half, 0, 0)),
            pl.BlockSpec((1, 1, D), lambda i: (i // half, 0, 0)),
        ],
        out_specs=pl.BlockSpec((BLK, D), out_map),
        out_shape=jax.ShapeDtypeStruct((2 * N_NODE, D), jnp.float32),
    )(s, cnt.reshape(2 * N_NODE, 1), x, wn, ws, b.reshape(2, 1, D))


def kernel(x_user, x_item, w_neigh_u2i_1, w_self_u2i_1, b_u2i_1,
           w_neigh_i2u_1, w_self_i2u_1, b_i2u_1,
           w_neigh_u2i_2, w_self_u2i_2, b_u2i_2,
           w_neigh_i2u_2, w_self_i2u_2, b_i2u_2,
           edge_index_u2i, edge_index_i2u):
    su = edge_index_u2i.astype(jnp.int32)
    si = edge_index_i2u.astype(jnp.int32)
    sidx_u = (su[0] + N_NODE).reshape(NS, NCH, K)
    didx_u = su[1].reshape(NS, NCH, K)
    sidx_i = si[0].reshape(NS, NCH, K)
    didx_i = si[1].reshape(NS, NCH, K)
    x_all = jnp.concatenate([x_item, x_user])
    wn1 = jnp.stack([w_neigh_u2i_1, w_neigh_i2u_1])
    ws1 = jnp.stack([w_self_u2i_1, w_self_i2u_1])
    b1 = jnp.stack([b_u2i_1, b_i2u_1])
    wn2 = jnp.stack([w_neigh_u2i_2, w_neigh_i2u_2])
    ws2 = jnp.stack([w_self_u2i_2, w_self_i2u_2])
    b2 = jnp.stack([b_u2i_2, b_i2u_2])

    s1, c1 = _sc_aggregate_l1(x_all, sidx_u, didx_u, sidx_i, didx_i)
    h_all = _tc_linear(s1, c1, x_all, wn1, ws1, b1,
                       relu=True, rotate_out=False)

    (s2,) = _sc_aggregate_l2(h_all, sidx_u, didx_u, sidx_i, didx_i)
    return _tc_linear(s2, c1, h_all, wn2, ws2, b2,
                      relu=False, rotate_out=True)
